# baseline (speedup 1.0000x reference)
import functools

import jax
import jax.numpy as jnp
from jax import lax
from jax.experimental import pallas as pl
from jax.experimental.pallas import tpu as pltpu
from jax.experimental.pallas import tpu_sc as plsc

NUM_CLASSES = 1000
BATCH = 16384
LANES = 16
NUM_WORKERS = 32
ROWS_PER_W = BATCH // NUM_WORKERS
FLAT = BATCH * NUM_CLASSES
CHUNK = 64 * NUM_CLASSES
CHUNKS_PER_W = ROWS_PER_W * NUM_CLASSES // CHUNK
IDX_ROWS = ROWS_PER_W // 128

_mesh = plsc.VectorSubcoreMesh(core_axis_name="c", subcore_axis_name="s")


@functools.partial(
    pl.kernel,
    out_type=jax.ShapeDtypeStruct((FLAT,), jnp.float32),
    mesh=_mesh,
    scratch_types=[
        pltpu.VMEM((CHUNK,), jnp.float32),
        pltpu.VMEM((IDX_ROWS, 128), jnp.int32),
        pltpu.VMEM((IDX_ROWS, 128), jnp.int32),
        pltpu.VMEM((IDX_ROWS, 128), jnp.float32),
        pltpu.SemaphoreType.DMA,
        pltpu.SemaphoreType.DMA,
    ],
)
def _onehot_body(labels_hbm, out_hbm, tmpl, lab_v, idx_v, val_v, sem_lin, sem_sct):
    cid = lax.axis_index("c")
    sid = lax.axis_index("s")
    wid = sid * 2 + cid
    row0 = wid * ROWS_PER_W
    base = row0 * NUM_CLASSES

    pltpu.sync_copy(labels_hbm.at[pl.ds(wid * IDX_ROWS, IDX_ROWS)], lab_v)

    def fill_body(i, carry):
        tmpl[pl.ds(i * LANES, LANES)] = jnp.full((LANES,), -10.0, jnp.float32)
        return carry

    lax.fori_loop(0, CHUNK // LANES, fill_body, 0, unroll=8)

    handles = [
        pltpu.async_copy(tmpl, out_hbm.at[pl.ds(base + c * CHUNK, CHUNK)], sem_lin)
        for c in range(CHUNKS_PER_W)
    ]

    iota16 = lax.iota(jnp.int32, LANES)
    for j in range(IDX_ROWS):
        for g in range(128 // LANES):
            lab16 = lab_v[j, pl.ds(g * LANES, LANES)]
            r16 = iota16 + (row0 + j * 128 + g * LANES)
            idx_v[j, pl.ds(g * LANES, LANES)] = r16 * NUM_CLASSES + lab16
            val_v[j, pl.ds(g * LANES, LANES)] = jnp.full((LANES,), 10.0, jnp.float32)

    for h in handles:
        h.wait()

    scatters = [
        pltpu.async_copy(val_v.at[j], out_hbm.at[idx_v.at[j]], sem_sct)
        for j in range(IDX_ROWS)
    ]
    for h in scatters:
        h.wait()


def kernel(input_ids, dummy):
    labels = input_ids[:, 0].astype(jnp.int32).reshape(BATCH // 128, 128)
    out = _onehot_body(labels)
    return out.reshape(BATCH, NUM_CLASSES)


# speedup vs baseline: 1.7701x; 1.7701x over previous
import functools

import jax
import jax.numpy as jnp
from jax import lax
from jax.experimental import pallas as pl
from jax.experimental.pallas import tpu as pltpu
from jax.experimental.pallas import tpu_sc as plsc

NUM_CLASSES = 1000
BATCH = 16384
LANES = 16
NUM_WORKERS = 32
ROWS_PER_W = BATCH // NUM_WORKERS
CHUNK_ROWS = 32
CHUNK_PAIRS = ROWS_PER_W // (2 * CHUNK_ROWS)

_mesh = plsc.VectorSubcoreMesh(core_axis_name="c", subcore_axis_name="s")


@functools.partial(
    pl.kernel,
    out_type=jax.ShapeDtypeStruct((BATCH, NUM_CLASSES), jnp.float32),
    mesh=_mesh,
    scratch_types=[
        pltpu.VMEM((CHUNK_ROWS, NUM_CLASSES), jnp.float32),
        pltpu.VMEM((CHUNK_ROWS, NUM_CLASSES), jnp.float32),
        pltpu.VMEM((ROWS_PER_W,), jnp.int32),
        pltpu.SemaphoreType.DMA,
        pltpu.SemaphoreType.DMA,
    ],
)
def _onehot_body(labels_hbm, out_hbm, tmpl_a, tmpl_b, lab_v, sem_a, sem_b):
    cid = lax.axis_index("c")
    sid = lax.axis_index("s")
    wid = sid * 2 + cid
    row0 = wid * ROWS_PER_W

    pltpu.sync_copy(labels_hbm.at[pl.ds(wid * ROWS_PER_W, ROWS_PER_W)], lab_v)

    minus_ten = jnp.full((LANES,), -10.0, jnp.float32)
    iota16 = lax.iota(jnp.int32, LANES)
    col_groups = [k * LANES for k in range(NUM_CLASSES // LANES)] + [NUM_CLASSES - LANES]

    def fill(tmpl):
        def fill_row(r, carry):
            for c0 in col_groups:
                tmpl[r, pl.ds(c0, LANES)] = minus_ten
            return carry
        lax.fori_loop(0, CHUNK_ROWS, fill_row, 0)

    fill(tmpl_a)
    fill(tmpl_b)

    def place(c, tmpl):
        for g in range(CHUNK_ROWS // LANES):
            lab16 = lab_v[pl.ds(c * CHUNK_ROWS + g * LANES, LANES)]
            for e in range(LANES):
                lab = lab16[e]
                val = jnp.where(iota16 == (lab & (LANES - 1)), 10.0, -10.0
                                ).astype(jnp.float32)
                col0 = pl.multiple_of((lab >> 4) << 4, LANES)
                tmpl[g * LANES + e, pl.ds(col0, LANES)] = val

    def restore(c, tmpl):
        for g in range(CHUNK_ROWS // LANES):
            lab16 = lab_v[pl.ds(c * CHUNK_ROWS + g * LANES, LANES)]
            for e in range(LANES):
                lab = lab16[e]
                col0 = pl.multiple_of((lab >> 4) << 4, LANES)
                tmpl[g * LANES + e, pl.ds(col0, LANES)] = minus_ten

    def pair_body(i, carry):
        for slot, (tmpl, sem) in enumerate(((tmpl_a, sem_a), (tmpl_b, sem_b))):
            c = 2 * i + slot

            @pl.when(i > 0)
            def _wait_and_restore():
                pltpu.make_async_copy(
                    tmpl, out_hbm.at[pl.ds(0, CHUNK_ROWS)], sem).wait()
                restore(c - 2, tmpl)

            place(c, tmpl)
            pltpu.async_copy(
                tmpl, out_hbm.at[pl.ds(row0 + c * CHUNK_ROWS, CHUNK_ROWS)], sem)
        return carry

    lax.fori_loop(0, CHUNK_PAIRS, pair_body, 0)

    pltpu.make_async_copy(tmpl_a, out_hbm.at[pl.ds(0, CHUNK_ROWS)], sem_a).wait()
    pltpu.make_async_copy(tmpl_b, out_hbm.at[pl.ds(0, CHUNK_ROWS)], sem_b).wait()


def kernel(input_ids, dummy):
    labels = input_ids[:, 0].astype(jnp.int32)
    return _onehot_body(labels)
